# Initial kernel scaffold; baseline (speedup 1.0000x reference)
import jax
import jax.numpy as jnp
from jax.experimental import pallas as pl


def kernel(x, emb_table):
    raise NotImplementedError("write your pallas kernel here")


# speedup vs baseline: 3.1136x; 3.1136x over previous
import jax
import jax.numpy as jnp
from jax.experimental import pallas as pl

_MAX_BOXES = 100
_EMB_DIM = 256


def _prep_body(emb_ref, target_ref, pe_ref):
    target_ref[...] = jnp.zeros(target_ref.shape, target_ref.dtype)
    pe_ref[...] = jnp.broadcast_to(emb_ref[...], pe_ref.shape)


def kernel(x, emb_table):
    b = x.shape[0]
    bb = 16
    out_shape = jax.ShapeDtypeStruct((b, _MAX_BOXES, _EMB_DIM), jnp.float32)
    target, target_pe = pl.pallas_call(
        _prep_body,
        grid=(b // bb,),
        in_specs=[pl.BlockSpec((_MAX_BOXES, _EMB_DIM), lambda i: (0, 0))],
        out_specs=[
            pl.BlockSpec((bb, _MAX_BOXES, _EMB_DIM), lambda i: (i, 0, 0)),
            pl.BlockSpec((bb, _MAX_BOXES, _EMB_DIM), lambda i: (i, 0, 0)),
        ],
        out_shape=[out_shape, out_shape],
    )(emb_table)
    return (target, target_pe)
